# baseline (speedup 1.0000x reference)
import functools

import jax
import jax.numpy as jnp
import numpy as np
from jax import lax
from jax.experimental import pallas as pl
from jax.experimental.pallas import tpu as pltpu
from jax.experimental.pallas import tpu_sc as plsc

NN = 10000
EE = 320000
FF = 128
HH = 128
RR = 20
CC = 8

NC = 2
NS = 16
LL = 16
BB = 80
CNT_PAD = 204800

_mesh = plsc.VectorSubcoreMesh(core_axis_name="c", subcore_axis_name="s")


EPC = EE // NS
EPT = EE // NC // NS
NCH = EPT // BB


def _prep(src, dst, et, zflat):
    @functools.partial(
        pl.kernel,
        out_type=(jax.ShapeDtypeStruct((EE,), jnp.int32),
                  jax.ShapeDtypeStruct((EE,), jnp.float32)),
        mesh=_mesh,
        scratch_types=[
            pltpu.VMEM_SHARED((CNT_PAD,), jnp.float32),
            pltpu.VMEM((BB,), jnp.float32),
            pltpu.VMEM((EPC,), jnp.int32),
            pltpu.VMEM((EPC,), jnp.int32),
            pltpu.VMEM((EPT,), jnp.int32),
            pltpu.VMEM((EPT,), jnp.int32),
            pltpu.VMEM((EPT,), jnp.float32),
            pltpu.VMEM((EPT,), jnp.float32),
            pltpu.VMEM((BB,), jnp.int32),
            pltpu.VMEM((BB,), jnp.int32),
            pltpu.SemaphoreType.DMA,
            pltpu.SemaphoreType.DMA,
        ],
    )
    def k(src_h, dst_h, et_h, z_h, gidx_h, scale_h, cnt_sh,
          ones, dstall, etall, srcall, gidxall, cvalall, scaleall,
          segb0, segb1, sem0, sem1):
        c = lax.axis_index("c")
        s = lax.axis_index("s")
        npt = CNT_PAD // NS
        pltpu.sync_copy(z_h.at[pl.ds(s * npt, npt)], cnt_sh.at[pl.ds(s * npt, npt)])
        for i in range(BB // LL):
            ones[pl.ds(i * LL, LL)] = jnp.full((LL,), 1.0, jnp.float32)
        plsc.subcore_barrier()

        segb = (segb0, segb1)
        sems = (sem0, sem1)

        cbase = pl.multiple_of(s * EPC, 8)
        pltpu.sync_copy(dst_h.at[pl.ds(cbase, EPC)], dstall)
        pltpu.sync_copy(et_h.at[pl.ds(cbase, EPC)], etall)

        def fill_segb(kk, b):
            for i in range(BB // LL):
                o = kk * BB + i * LL
                segb[b][pl.ds(i * LL, LL)] = (
                    dstall[pl.ds(o, LL)] * RR + etall[pl.ds(o, LL)])

        fill_segb(0, 0)
        pltpu.async_copy(ones, cnt_sh.at[segb[0]], sems[0], add=True)
        fill_segb(1, 1)
        pltpu.async_copy(ones, cnt_sh.at[segb[1]], sems[1], add=True)

        def count2_tail(k2, _):
            kk = 2 + k2 * 2
            pltpu.make_async_copy(ones, cnt_sh.at[segb[0]], sems[0]).wait()
            fill_segb(kk, 0)
            pltpu.async_copy(ones, cnt_sh.at[segb[0]], sems[0], add=True)
            pltpu.make_async_copy(ones, cnt_sh.at[segb[1]], sems[1]).wait()
            fill_segb(kk + 1, 1)
            pltpu.async_copy(ones, cnt_sh.at[segb[1]], sems[1], add=True)
            return 0

        lax.fori_loop(0, (EPC // BB - 2) // 2, count2_tail, 0)
        pltpu.make_async_copy(ones, cnt_sh.at[segb[0]], sems[0]).wait()
        pltpu.make_async_copy(ones, cnt_sh.at[segb[1]], sems[1]).wait()
        plsc.subcore_barrier()

        ebase = pl.multiple_of(c * (EE // NC) + s * EPT, 8)
        pltpu.sync_copy(src_h.at[pl.ds(ebase, EPT)], srcall)
        pltpu.sync_copy(dst_h.at[pl.ds(ebase, EPT)], dstall.at[pl.ds(0, EPT)])
        pltpu.sync_copy(et_h.at[pl.ds(ebase, EPT)], etall.at[pl.ds(0, EPT)])

        def gidx_iter(i, _):
            o = pl.multiple_of(i * LL, LL)
            gidxall[pl.ds(o, LL)] = (
                etall[pl.ds(o, LL)] * NN + srcall[pl.ds(o, LL)])
            return 0

        lax.fori_loop(0, EPT // LL, gidx_iter, 0)
        pltpu.sync_copy(gidxall, gidx_h.at[pl.ds(ebase, EPT)])

        def seg_iter(i, _):
            o = pl.multiple_of(i * LL, LL)
            srcall[pl.ds(o, LL)] = (
                dstall[pl.ds(o, LL)] * RR + etall[pl.ds(o, LL)])
            return 0

        lax.fori_loop(0, EPT // LL, seg_iter, 0)

        def cnt_gather(k2, _):
            kk = k2 * 2
            cp0 = pltpu.async_copy(
                cnt_sh.at[srcall.at[pl.ds(kk * BB, BB)]],
                cvalall.at[pl.ds(kk * BB, BB)], sems[0])
            cp1 = pltpu.async_copy(
                cnt_sh.at[srcall.at[pl.ds(kk * BB + BB, BB)]],
                cvalall.at[pl.ds(kk * BB + BB, BB)], sems[1])
            cp0.wait()
            cp1.wait()
            return 0

        lax.fori_loop(0, NCH // 2, cnt_gather, 0)
        pltpu.async_copy(
            cnt_sh.at[srcall.at[pl.ds((NCH - 1) * BB, BB)]],
            cvalall.at[pl.ds((NCH - 1) * BB, BB)], sems[0]).wait()

        def scale_iter(i, _):
            o = pl.multiple_of(i * LL, LL)
            scaleall[pl.ds(o, LL)] = 1.0 / jnp.maximum(cvalall[pl.ds(o, LL)], 1.0)
            return 0

        lax.fori_loop(0, EPT // LL, scale_iter, 0)
        pltpu.sync_copy(scaleall, scale_h.at[pl.ds(ebase, EPT)])

    return k(src, dst, et, zflat)


def _agg(T, gidx, dst, scale, zeros_nh):
    @functools.partial(
        pl.kernel,
        out_type=jax.ShapeDtypeStruct((NC * NN, HH), jnp.float32),
        mesh=_mesh,
        compiler_params=pltpu.CompilerParams(use_tc_tiling_on_sc=False),
        scratch_types=[
            pltpu.VMEM_SHARED((NN, HH), jnp.float32),
            pltpu.VMEM((EPT,), jnp.int32),
            pltpu.VMEM((EPT,), jnp.float32),
            pltpu.VMEM((BB,), jnp.int32),
            pltpu.VMEM((BB,), jnp.int32),
            pltpu.VMEM((BB, HH // 2), jnp.int32),
            pltpu.VMEM((BB, HH // 2), jnp.int32),
            pltpu.VMEM((BB, HH), jnp.float32),
            pltpu.VMEM((BB, HH), jnp.float32),
            pltpu.SemaphoreType.DMA,
            pltpu.SemaphoreType.DMA,
            pltpu.SemaphoreType.DMA,
            pltpu.SemaphoreType.DMA,
        ],
    )
    def k(T_h, gidx_h, dst_h, scale_h, z_h, out_h, acc_sh,
          gidxall, scaleall, dstb0, dstb1, rw0, rw1, rf0, rf1,
          sd0, sd1, sg0, sg1):
        c = lax.axis_index("c")
        s = lax.axis_index("s")
        row0 = pl.multiple_of(s * 624, 8)

        @pl.when(s < NS - 1)
        def _():
            pltpu.sync_copy(z_h.at[pl.ds(row0, 624)], acc_sh.at[pl.ds(row0, 624)])

        @pl.when(s == NS - 1)
        def _():
            pltpu.sync_copy(z_h.at[pl.ds(row0, 640)], acc_sh.at[pl.ds(row0, 640)])

        plsc.subcore_barrier()

        ebase = pl.multiple_of(c * (EE // NC) + s * EPT, 8)
        pltpu.sync_copy(gidx_h.at[pl.ds(ebase, EPT)], gidxall)
        pltpu.sync_copy(scale_h.at[pl.ds(ebase, EPT)], scaleall)

        dstb = (dstb0, dstb1)
        rw = (rw0, rw1)
        rf = (rf0, rf1)
        sd = (sd0, sd1)
        sg = (sg0, sg1)

        def issue(kk, b):
            pltpu.async_copy(dst_h.at[pl.ds(ebase + kk * BB, BB)], dstb[b], sd[b])
            pltpu.async_copy(T_h.at[gidxall.at[pl.ds(kk * BB, BB)]], rw[b], sg[b])

        def wait(b):
            pltpu.make_async_copy(dst_h.at[pl.ds(ebase, BB)], dstb[b], sd[b]).wait()
            pltpu.make_async_copy(T_h.at[pl.ds(0, BB)], rw[b], sg[b]).wait()

        def compute(kk, b):
            rb = rw[b]
            rff = rf[b]
            for gblk in range(BB // LL):
                s16 = scaleall[pl.ds(kk * BB + gblk * LL, LL)]
                for i in range(LL):
                    e = gblk * LL + i
                    sv = s16[i]
                    for c4 in range(4):
                        w32 = rb[e, pl.ds(c4 * LL, LL)]
                        lo = lax.bitcast_convert_type(w32 << 16, jnp.float32)
                        hi = lax.bitcast_convert_type(w32 & jnp.int32(-65536), jnp.float32)
                        rff[e, pl.ds(c4 * 2 * LL, LL)] = lo * sv
                        rff[e, pl.ds(c4 * 2 * LL + LL, LL)] = hi * sv
            pltpu.sync_copy(rff, acc_sh.at[dstb[b]], add=True)

        issue(0, 0)

        def body(k2, _):
            kk = k2 * 2

            @pl.when(kk + 1 < NCH)
            def _():
                issue(kk + 1, 1)

            wait(0)
            compute(kk, 0)

            @pl.when(kk + 2 < NCH)
            def _():
                issue(kk + 2, 0)

            @pl.when(kk + 1 < NCH)
            def _():
                wait(1)
                compute(kk + 1, 1)

            return 0

        lax.fori_loop(0, (NCH + 1) // 2, body, 0)
        plsc.subcore_barrier()
        obase = pl.multiple_of(c * NN + s * 624, 8)

        @pl.when(s < NS - 1)
        def _():
            pltpu.sync_copy(acc_sh.at[pl.ds(row0, 624)], out_h.at[pl.ds(obase, 624)])

        @pl.when(s == NS - 1)
        def _():
            pltpu.sync_copy(acc_sh.at[pl.ds(row0, 640)], out_h.at[pl.ds(obase, 640)])

    return k(T, gidx, dst, scale, zeros_nh)


def _mm(x, wall):
    nb = 5
    bn = NN // nb

    def body(x_ref, w_ref, o_ref):
        y = jnp.dot(x_ref[...], w_ref[0], preferred_element_type=jnp.float32)
        u1 = lax.bitcast_convert_type(y[:, :HH // 2], jnp.uint32)
        u2 = lax.bitcast_convert_type(y[:, HH // 2:], jnp.uint32)
        w32 = ((u1 + 0x8000) >> 16) | ((u2 + 0x8000) & jnp.uint32(0xFFFF0000))
        o_ref[0] = lax.bitcast_convert_type(w32, jnp.int32)

    return pl.pallas_call(
        body,
        grid=(nb, RR),
        in_specs=[pl.BlockSpec((bn, FF), lambda j, r: (j, 0)),
                  pl.BlockSpec((1, FF, HH), lambda j, r: (r, 0, 0))],
        out_specs=pl.BlockSpec((1, bn, HH // 2), lambda j, r: (r, j, 0)),
        out_shape=jax.ShapeDtypeStruct((RR, NN, HH // 2), jnp.int32),
    )(x, wall)


def _post(p0, p1, h, rootw, bvec, g, be, wf=None, bfin=None):
    nb = 10
    bn = NN // nb
    final = wf is not None
    oc = CC if final else HH

    def body(*refs):
        if final:
            p0_r, p1_r, h_r, rw_r, b_r, g_r, be_r, wf_r, bf_r, o_ref, s1, s2, yall = refs
        else:
            p0_r, p1_r, h_r, rw_r, b_r, g_r, be_r, o_ref, s1, s2, yall = refs
        p = pl.program_id(0)
        j = pl.program_id(1)

        @pl.when(p == 0)
        def _():
            y = (p0_r[...] + p1_r[...]
                 + jnp.dot(h_r[...], rw_r[...], preferred_element_type=jnp.float32)
                 + b_r[...])
            yall[pl.ds(j * bn, bn), :] = y

            @pl.when(j == 0)
            def _():
                s1[...] = jnp.zeros_like(s1)
                s2[...] = jnp.zeros_like(s2)
            s1[...] += jnp.sum(y, axis=0, keepdims=True)
            s2[...] += jnp.sum(y * y, axis=0, keepdims=True)

        @pl.when(p == 1)
        def _():
            y = yall[pl.ds(j * bn, bn), :]
            mu = s1[...] / NN
            var = s2[...] / NN - mu * mu
            z = jnp.maximum((y - mu) * lax.rsqrt(var + 1e-5) * g_r[...] + be_r[...], 0.0)
            if final:
                o_ref[...] = jnp.dot(z, wf_r[...], preferred_element_type=jnp.float32) + bf_r[...]
            else:
                o_ref[...] = z

    row_spec = pl.BlockSpec((bn, HH), lambda p, j: (j * (1 - p), 0))
    vec_spec = pl.BlockSpec((1, HH), lambda p, j: (0, 0))
    in_specs = [row_spec, row_spec,
                pl.BlockSpec((bn, FF), lambda p, j: (j * (1 - p), 0)),
                pl.BlockSpec((FF, HH), lambda p, j: (0, 0)),
                vec_spec, vec_spec, vec_spec]
    args = [p0, p1, h, rootw, bvec.reshape(1, HH), g.reshape(1, HH), be.reshape(1, HH)]
    if final:
        in_specs += [pl.BlockSpec((HH, CC), lambda p, j: (0, 0)),
                     pl.BlockSpec((1, CC), lambda p, j: (0, 0))]
        args += [wf, bfin.reshape(1, CC)]

    return pl.pallas_call(
        body,
        grid=(2, nb),
        in_specs=in_specs,
        out_specs=pl.BlockSpec((bn, oc), lambda p, j: (j, 0)),
        out_shape=jax.ShapeDtypeStruct((NN, oc), jnp.float32),
        scratch_shapes=[pltpu.VMEM((1, HH), jnp.float32),
                        pltpu.VMEM((1, HH), jnp.float32),
                        pltpu.VMEM((NN, HH), jnp.float32)],
    )(*args)


def kernel(x, edge_index, edge_attr, batch, W1, root1, b1, g1, be1,
           W2, root2, b2, g2, be2, Wf, bf):
    src = edge_index[0]
    dst = edge_index[1]
    et = edge_attr

    zeros_nh = jnp.zeros((NN, HH), jnp.float32)
    zflat = jnp.zeros((CNT_PAD,), jnp.float32)
    gidx, scale = _prep(src, dst, et, zflat)

    perm = np.empty((HH,), np.int32)
    for c4 in range(HH // 32):
        for i in range(16):
            perm[16 * c4 + i] = 32 * c4 + i
            perm[64 + 16 * c4 + i] = 32 * c4 + 16 + i
    perm = jnp.asarray(perm)

    h = x
    for idx, (w, root, bvec, g, be) in enumerate(
            ((W1, root1, b1, g1, be1), (W2, root2, b2, g2, be2))):
        wall = w[:, :, perm].astype(jnp.bfloat16)
        t32 = _mm(h.astype(jnp.bfloat16), wall).reshape(RR * NN, HH // 2)
        parts = _agg(t32, gidx, dst, scale, zeros_nh)
        if idx == 0:
            h = _post(parts[:NN], parts[NN:], h, root, bvec, g, be)
        else:
            h = _post(parts[:NN], parts[NN:], h, root, bvec, g, be, Wf, bf)
    return h


# speedup vs baseline: 1.0137x; 1.0137x over previous
import functools

import jax
import jax.numpy as jnp
import numpy as np
from jax import lax
from jax.experimental import pallas as pl
from jax.experimental.pallas import tpu as pltpu
from jax.experimental.pallas import tpu_sc as plsc

NN = 10000
EE = 320000
FF = 128
HH = 128
RR = 20
CC = 8

NC = 2
NS = 16
LL = 16
BB = 80
CNT_PAD = 204800

_mesh = plsc.VectorSubcoreMesh(core_axis_name="c", subcore_axis_name="s")


EPC = EE // NS
EPT = EE // NC // NS
NCH = EPT // BB


def _prep(src, dst, et, zflat):
    @functools.partial(
        pl.kernel,
        out_type=(jax.ShapeDtypeStruct((EE,), jnp.int32),
                  jax.ShapeDtypeStruct((EE,), jnp.float32)),
        mesh=_mesh,
        scratch_types=[
            pltpu.VMEM_SHARED((CNT_PAD,), jnp.float32),
            pltpu.VMEM((BB,), jnp.float32),
            pltpu.VMEM((EPC,), jnp.int32),
            pltpu.VMEM((EPC,), jnp.int32),
            pltpu.VMEM((EPT,), jnp.int32),
            pltpu.VMEM((EPT,), jnp.int32),
            pltpu.VMEM((EPT,), jnp.float32),
            pltpu.VMEM((EPT,), jnp.float32),
            pltpu.VMEM((BB,), jnp.int32),
            pltpu.VMEM((BB,), jnp.int32),
            pltpu.SemaphoreType.DMA,
            pltpu.SemaphoreType.DMA,
        ],
    )
    def k(src_h, dst_h, et_h, z_h, gidx_h, scale_h, cnt_sh,
          ones, dstall, etall, srcall, gidxall, cvalall, scaleall,
          segb0, segb1, sem0, sem1):
        c = lax.axis_index("c")
        s = lax.axis_index("s")
        npt = CNT_PAD // NS
        pltpu.sync_copy(z_h.at[pl.ds(s * npt, npt)], cnt_sh.at[pl.ds(s * npt, npt)])
        for i in range(BB // LL):
            ones[pl.ds(i * LL, LL)] = jnp.full((LL,), 1.0, jnp.float32)
        plsc.subcore_barrier()

        segb = (segb0, segb1)
        sems = (sem0, sem1)

        cbase = pl.multiple_of(s * EPC, 8)
        pltpu.sync_copy(dst_h.at[pl.ds(cbase, EPC)], dstall)
        pltpu.sync_copy(et_h.at[pl.ds(cbase, EPC)], etall)

        def fill_segb(kk, b):
            for i in range(BB // LL):
                o = kk * BB + i * LL
                segb[b][pl.ds(i * LL, LL)] = (
                    dstall[pl.ds(o, LL)] * RR + etall[pl.ds(o, LL)])

        fill_segb(0, 0)
        pltpu.async_copy(ones, cnt_sh.at[segb[0]], sems[0], add=True)
        fill_segb(1, 1)
        pltpu.async_copy(ones, cnt_sh.at[segb[1]], sems[1], add=True)

        def count2_tail(k2, _):
            kk = 2 + k2 * 2
            pltpu.make_async_copy(ones, cnt_sh.at[segb[0]], sems[0]).wait()
            fill_segb(kk, 0)
            pltpu.async_copy(ones, cnt_sh.at[segb[0]], sems[0], add=True)
            pltpu.make_async_copy(ones, cnt_sh.at[segb[1]], sems[1]).wait()
            fill_segb(kk + 1, 1)
            pltpu.async_copy(ones, cnt_sh.at[segb[1]], sems[1], add=True)
            return 0

        lax.fori_loop(0, (EPC // BB - 2) // 2, count2_tail, 0)
        pltpu.make_async_copy(ones, cnt_sh.at[segb[0]], sems[0]).wait()
        pltpu.make_async_copy(ones, cnt_sh.at[segb[1]], sems[1]).wait()
        plsc.subcore_barrier()

        ebase = pl.multiple_of(c * (EE // NC) + s * EPT, 8)
        pltpu.sync_copy(src_h.at[pl.ds(ebase, EPT)], srcall)
        pltpu.sync_copy(dst_h.at[pl.ds(ebase, EPT)], dstall.at[pl.ds(0, EPT)])
        pltpu.sync_copy(et_h.at[pl.ds(ebase, EPT)], etall.at[pl.ds(0, EPT)])

        def gidx_iter(i, _):
            o = pl.multiple_of(i * LL, LL)
            gidxall[pl.ds(o, LL)] = (
                etall[pl.ds(o, LL)] * NN + srcall[pl.ds(o, LL)])
            return 0

        lax.fori_loop(0, EPT // LL, gidx_iter, 0)
        pltpu.sync_copy(gidxall, gidx_h.at[pl.ds(ebase, EPT)])

        def seg_iter(i, _):
            o = pl.multiple_of(i * LL, LL)
            srcall[pl.ds(o, LL)] = (
                dstall[pl.ds(o, LL)] * RR + etall[pl.ds(o, LL)])
            return 0

        lax.fori_loop(0, EPT // LL, seg_iter, 0)

        def cnt_gather(k2, _):
            kk = k2 * 2
            cp0 = pltpu.async_copy(
                cnt_sh.at[srcall.at[pl.ds(kk * BB, BB)]],
                cvalall.at[pl.ds(kk * BB, BB)], sems[0])
            cp1 = pltpu.async_copy(
                cnt_sh.at[srcall.at[pl.ds(kk * BB + BB, BB)]],
                cvalall.at[pl.ds(kk * BB + BB, BB)], sems[1])
            cp0.wait()
            cp1.wait()
            return 0

        lax.fori_loop(0, NCH // 2, cnt_gather, 0)
        pltpu.async_copy(
            cnt_sh.at[srcall.at[pl.ds((NCH - 1) * BB, BB)]],
            cvalall.at[pl.ds((NCH - 1) * BB, BB)], sems[0]).wait()

        def scale_iter(i, _):
            o = pl.multiple_of(i * LL, LL)
            scaleall[pl.ds(o, LL)] = 1.0 / jnp.maximum(cvalall[pl.ds(o, LL)], 1.0)
            return 0

        lax.fori_loop(0, EPT // LL, scale_iter, 0)
        pltpu.sync_copy(scaleall, scale_h.at[pl.ds(ebase, EPT)])

    return k(src, dst, et, zflat)


def _agg(T, gidx, dst, scale, zeros_nh):
    @functools.partial(
        pl.kernel,
        out_type=jax.ShapeDtypeStruct((NC * NN, HH), jnp.float32),
        mesh=_mesh,
        compiler_params=pltpu.CompilerParams(use_tc_tiling_on_sc=False),
        scratch_types=[
            pltpu.VMEM_SHARED((NN, HH), jnp.float32),
            pltpu.VMEM((EPT,), jnp.int32),
            pltpu.VMEM((EPT,), jnp.float32),
            pltpu.VMEM((BB,), jnp.int32),
            pltpu.VMEM((BB,), jnp.int32),
            pltpu.VMEM((BB, HH // 2), jnp.int32),
            pltpu.VMEM((BB, HH // 2), jnp.int32),
            pltpu.VMEM((BB, HH), jnp.float32),
            pltpu.VMEM((BB, HH), jnp.float32),
            pltpu.SemaphoreType.DMA,
            pltpu.SemaphoreType.DMA,
            pltpu.SemaphoreType.DMA,
            pltpu.SemaphoreType.DMA,
        ],
    )
    def k(T_h, gidx_h, dst_h, scale_h, z_h, out_h, acc_sh,
          gidxall, scaleall, dstb0, dstb1, rw0, rw1, rf0, rf1,
          sd0, sd1, sg0, sg1):
        c = lax.axis_index("c")
        s = lax.axis_index("s")
        row0 = pl.multiple_of(s * 624, 8)

        @pl.when(s < NS - 1)
        def _():
            pltpu.sync_copy(z_h.at[pl.ds(row0, 624)], acc_sh.at[pl.ds(row0, 624)])

        @pl.when(s == NS - 1)
        def _():
            pltpu.sync_copy(z_h.at[pl.ds(row0, 640)], acc_sh.at[pl.ds(row0, 640)])

        plsc.subcore_barrier()

        ebase = pl.multiple_of(c * (EE // NC) + s * EPT, 8)
        pltpu.sync_copy(gidx_h.at[pl.ds(ebase, EPT)], gidxall)
        pltpu.sync_copy(scale_h.at[pl.ds(ebase, EPT)], scaleall)

        dstb = (dstb0, dstb1)
        rw = (rw0, rw1)
        rf = (rf0, rf1)
        sd = (sd0, sd1)
        sg = (sg0, sg1)

        def issue(kk, b):
            pltpu.async_copy(dst_h.at[pl.ds(ebase + kk * BB, BB)], dstb[b], sd[b])
            pltpu.async_copy(T_h.at[gidxall.at[pl.ds(kk * BB, BB)]], rw[b], sg[b])

        def wait(b):
            pltpu.make_async_copy(dst_h.at[pl.ds(ebase, BB)], dstb[b], sd[b]).wait()
            pltpu.make_async_copy(T_h.at[pl.ds(0, BB)], rw[b], sg[b]).wait()

        def compute(kk, b):
            rb = rw[b]
            rff = rf[b]
            for gblk in range(BB // LL):
                s16 = scaleall[pl.ds(kk * BB + gblk * LL, LL)]
                for i in range(LL):
                    e = gblk * LL + i
                    sv = s16[i]
                    for c4 in range(4):
                        w32 = rb[e, pl.ds(c4 * LL, LL)]
                        lo = lax.bitcast_convert_type(w32 << 16, jnp.float32)
                        hi = lax.bitcast_convert_type(w32 & jnp.int32(-65536), jnp.float32)
                        rff[e, pl.ds(c4 * 2 * LL, LL)] = lo * sv
                        rff[e, pl.ds(c4 * 2 * LL + LL, LL)] = hi * sv
            pltpu.sync_copy(rff, acc_sh.at[dstb[b]], add=True)

        issue(0, 0)

        def body(k2, _):
            kk = k2 * 2

            @pl.when(kk + 1 < NCH)
            def _():
                issue(kk + 1, 1)

            wait(0)
            compute(kk, 0)

            @pl.when(kk + 2 < NCH)
            def _():
                issue(kk + 2, 0)

            @pl.when(kk + 1 < NCH)
            def _():
                wait(1)
                compute(kk + 1, 1)

            return 0

        lax.fori_loop(0, (NCH + 1) // 2, body, 0)
        plsc.subcore_barrier()
        obase = pl.multiple_of(c * NN + s * 624, 8)

        @pl.when(s < NS - 1)
        def _():
            pltpu.sync_copy(acc_sh.at[pl.ds(row0, 624)], out_h.at[pl.ds(obase, 624)])

        @pl.when(s == NS - 1)
        def _():
            pltpu.sync_copy(acc_sh.at[pl.ds(row0, 640)], out_h.at[pl.ds(obase, 640)])

    return k(T, gidx, dst, scale, zeros_nh)


def _mm(x, wall):
    nb = 5
    bn = NN // nb

    def body(x_ref, w_ref, o_ref):
        y = jnp.dot(x_ref[...], w_ref[0], preferred_element_type=jnp.float32)
        u1 = lax.bitcast_convert_type(y[:, :HH // 2], jnp.uint32)
        u2 = lax.bitcast_convert_type(y[:, HH // 2:], jnp.uint32)
        w32 = ((u1 + 0x8000) >> 16) | ((u2 + 0x8000) & jnp.uint32(0xFFFF0000))
        o_ref[0] = lax.bitcast_convert_type(w32, jnp.int32)

    return pl.pallas_call(
        body,
        grid=(nb, RR),
        in_specs=[pl.BlockSpec((bn, FF), lambda j, r: (j, 0)),
                  pl.BlockSpec((1, FF, HH), lambda j, r: (r, 0, 0))],
        out_specs=pl.BlockSpec((1, bn, HH // 2), lambda j, r: (r, j, 0)),
        out_shape=jax.ShapeDtypeStruct((RR, NN, HH // 2), jnp.int32),
    )(x, wall)


def _post(p0, p1, h, rootw, bvec, g, be, wf=None, bfin=None):
    nb = 10
    bn = NN // nb
    final = wf is not None
    oc = CC if final else HH

    def body(*refs):
        if final:
            p0_r, p1_r, h_r, rw_r, b_r, g_r, be_r, wf_r, bf_r, o_ref, s1, s2, yall = refs
        else:
            p0_r, p1_r, h_r, rw_r, b_r, g_r, be_r, o_ref, s1, s2, yall = refs
        p = pl.program_id(0)
        j = pl.program_id(1)

        @pl.when(p == 0)
        def _():
            y = (p0_r[...] + p1_r[...]
                 + jnp.dot(h_r[...], rw_r[...], preferred_element_type=jnp.float32)
                 + b_r[...])
            yall[pl.ds(j * bn, bn), :] = y

            @pl.when(j == 0)
            def _():
                s1[...] = jnp.zeros_like(s1)
                s2[...] = jnp.zeros_like(s2)
            s1[...] += jnp.sum(y, axis=0, keepdims=True)
            s2[...] += jnp.sum(y * y, axis=0, keepdims=True)

        @pl.when(p == 1)
        def _():
            y = yall[pl.ds(j * bn, bn), :]
            mu = s1[...] / NN
            var = s2[...] / NN - mu * mu
            z = jnp.maximum((y - mu) * lax.rsqrt(var + 1e-5) * g_r[...] + be_r[...], 0.0)
            if final:
                o_ref[...] = jnp.dot(z, wf_r[...], preferred_element_type=jnp.float32) + bf_r[...]
            else:
                o_ref[...] = z.astype(jnp.bfloat16)

    row_spec = pl.BlockSpec((bn, HH), lambda p, j: (j * (1 - p), 0))
    vec_spec = pl.BlockSpec((1, HH), lambda p, j: (0, 0))
    in_specs = [row_spec, row_spec,
                pl.BlockSpec((bn, FF), lambda p, j: (j * (1 - p), 0)),
                pl.BlockSpec((FF, HH), lambda p, j: (0, 0)),
                vec_spec, vec_spec, vec_spec]
    args = [p0, p1, h, rootw, bvec.reshape(1, HH), g.reshape(1, HH), be.reshape(1, HH)]
    if final:
        in_specs += [pl.BlockSpec((HH, CC), lambda p, j: (0, 0)),
                     pl.BlockSpec((1, CC), lambda p, j: (0, 0))]
        args += [wf, bfin.reshape(1, CC)]

    return pl.pallas_call(
        body,
        grid=(2, nb),
        in_specs=in_specs,
        out_specs=pl.BlockSpec((bn, oc), lambda p, j: (j, 0)),
        out_shape=jax.ShapeDtypeStruct(
            (NN, oc), jnp.float32 if final else jnp.bfloat16),
        scratch_shapes=[pltpu.VMEM((1, HH), jnp.float32),
                        pltpu.VMEM((1, HH), jnp.float32),
                        pltpu.VMEM((NN, HH), jnp.float32)],
    )(*args)


def kernel(x, edge_index, edge_attr, batch, W1, root1, b1, g1, be1,
           W2, root2, b2, g2, be2, Wf, bf):
    src = edge_index[0]
    dst = edge_index[1]
    et = edge_attr

    zeros_nh = jnp.zeros((NN, HH), jnp.float32)
    zflat = jnp.zeros((CNT_PAD,), jnp.float32)
    gidx, scale = _prep(src, dst, et, zflat)

    perm = np.empty((HH,), np.int32)
    for c4 in range(HH // 32):
        for i in range(16):
            perm[16 * c4 + i] = 32 * c4 + i
            perm[64 + 16 * c4 + i] = 32 * c4 + 16 + i
    perm = jnp.asarray(perm)

    h = x
    for idx, (w, root, bvec, g, be) in enumerate(
            ((W1, root1, b1, g1, be1), (W2, root2, b2, g2, be2))):
        wall = w[:, :, perm].astype(jnp.bfloat16)
        hb = h.astype(jnp.bfloat16) if h.dtype != jnp.bfloat16 else h
        t32 = _mm(hb, wall).reshape(RR * NN, HH // 2)
        parts = _agg(t32, gidx, dst, scale, zeros_nh)
        if idx == 0:
            h = _post(parts[:NN], parts[NN:], hb, root.astype(jnp.bfloat16),
                      bvec, g, be)
        else:
            h = _post(parts[:NN], parts[NN:], hb, root.astype(jnp.bfloat16),
                      bvec, g, be, Wf, bf)
    return h
